# Initial kernel scaffold; baseline (speedup 1.0000x reference)
import jax
import jax.numpy as jnp
from jax.experimental import pallas as pl


def kernel(data, W1, b1, s1, gn1_w, gn1_b, gn1_ms, W2, b2, s2, s3, gn3_w, gn3_b, gn3_ms):
    raise NotImplementedError("write your pallas kernel here")


# speedup vs baseline: 26.2093x; 26.2093x over previous
import functools

import jax
import jax.numpy as jnp
from jax import lax
from jax.experimental import pallas as pl
from jax.experimental.pallas import tpu as pltpu
from jax.experimental.pallas import tpu_sc as plsc

_B = 8
_N = 2048
_D = 128
_NODES = _B * _N
_THR = 0.7
_RB = 512
_NBLK = 16
_BLK = _NODES // _NBLK
_NC, _NS = 2, 16
_EPW = _NODES // (_NC * _NS)
_HALF = _NODES // _NC
_IC = 128
_F32 = jnp.float32


def _edges_body(d_ref, tl_ref, w_ref, dis_ref):
    b = pl.program_id(0)
    X = d_ref[0]
    nchunks = _N // _RB

    def chunk(k, deg_acc):
        Xc = d_ref[0, pl.ds(k * _RB, _RB), :]
        S = lax.dot_general(Xc, X, (((1,), (1,)), ((), ())),
                            preferred_element_type=_F32)
        m = jnp.max(S, axis=1, keepdims=True)
        col = lax.broadcasted_iota(jnp.int32, (_RB, _N), 1)
        am = jnp.min(jnp.where(S == m, col, _N), axis=1)
        se = jnp.sum(jnp.exp(S - m), axis=1)
        wv = (1.0 / se > _THR).astype(_F32)
        deg_part = jnp.sum(
            jnp.where(am[:, None] == col, wv[:, None], 0.0), axis=0)
        tl_ref[0, 0, pl.ds(k * _RB, _RB)] = am + (b % (_B // _NC)) * _N
        w_ref[0, 0, pl.ds(k * _RB, _RB)] = wv
        return deg_acc + deg_part

    deg = lax.fori_loop(0, nchunks, chunk, jnp.zeros((_N,), _F32))
    dis_ref[0, 0, :] = lax.rsqrt(1.0 + deg)


_edges = pl.pallas_call(
    _edges_body,
    grid=(_B,),
    in_specs=[pl.BlockSpec((1, _N, _D), lambda b: (b, 0, 0))],
    out_specs=[
        pl.BlockSpec((1, 1, _N), lambda b: (b, 0, 0)),
        pl.BlockSpec((1, 1, _N), lambda b: (b, 0, 0)),
        pl.BlockSpec((1, 1, _N), lambda b: (b, 0, 0)),
    ],
    out_shape=[
        jax.ShapeDtypeStruct((_B, 1, _N), jnp.int32),
        jax.ShapeDtypeStruct((_B, 1, _N), _F32),
        jax.ShapeDtypeStruct((_B, 1, _N), _F32),
    ],
)


def _pre_body(x_ref, W_ref, dis_ref, w_ref, msg_ref, hd_ref):
    h = jnp.dot(x_ref[...], W_ref[...], preferred_element_type=_F32)
    hd = h * dis_ref[...]
    hd_ref[...] = hd
    msg_ref[...] = hd * w_ref[...]


_pre = pl.pallas_call(
    _pre_body,
    grid=(_NBLK,),
    in_specs=[
        pl.BlockSpec((_BLK, _D), lambda i: (i, 0)),
        pl.BlockSpec((_D, _D), lambda i: (0, 0)),
        pl.BlockSpec((_BLK, 1), lambda i: (i, 0)),
        pl.BlockSpec((_BLK, 1), lambda i: (i, 0)),
    ],
    out_specs=[
        pl.BlockSpec((_BLK, _D), lambda i: (i, 0)),
        pl.BlockSpec((_BLK, _D), lambda i: (i, 0)),
    ],
    out_shape=[
        jax.ShapeDtypeStruct((_NODES, _D), _F32),
        jax.ShapeDtypeStruct((_NODES, _D), _F32),
    ],
)


def _scatter_body(msg_hbm, hd_hbm, tl_hbm, out_hbm, shared, msg_v, idx_v):
    c = lax.axis_index("c")
    s = lax.axis_index("s")
    base = c * _HALF + s * _EPW
    pltpu.sync_copy(hd_hbm.at[pl.ds(base, _EPW)],
                    shared.at[pl.ds(s * _EPW, _EPW)])
    plsc.subcore_barrier()
    wid = c * _NS + s
    pltpu.sync_copy(tl_hbm.at[pl.ds(wid * (_EPW // _IC), _EPW // _IC)], idx_v)
    for j in range(_EPW // _IC):
        pltpu.sync_copy(msg_hbm.at[pl.ds(base + j * _IC, _IC)], msg_v)
        pltpu.sync_copy(msg_v, shared.at[idx_v.at[j]], add=True)
    plsc.subcore_barrier()
    pltpu.sync_copy(shared.at[pl.ds(s * _EPW, _EPW)],
                    out_hbm.at[pl.ds(base, _EPW)])


@functools.cache
def _get_scatter():
    return pl.kernel(
        _scatter_body,
        out_type=jax.ShapeDtypeStruct((_NODES, _D), _F32),
        mesh=plsc.VectorSubcoreMesh(core_axis_name="c", subcore_axis_name="s",
                                    num_cores=_NC, num_subcores=_NS),
        scratch_types=[
            pltpu.VMEM_SHARED((_HALF, _D), _F32),
            pltpu.VMEM((_IC, _D), _F32),
            pltpu.VMEM((_EPW // _IC, _IC), jnp.int32),
        ],
    )


def _scatter(msg, hd, tl):
    return _get_scatter()(msg, hd, tl)


def _msg_norm(h, x_norm, s):
    hn = jnp.sqrt(jnp.sum(h * h, axis=1, keepdims=True))
    return h / jnp.maximum(hn, 1e-12) * x_norm * s


def _accum_stats(i, st_ref, f):
    @pl.when(i == 0)
    def _():
        st_ref[...] = jnp.zeros_like(st_ref)

    st_ref[0:1, :] += jnp.sum(f, axis=0, keepdims=True)
    st_ref[1:2, :] += jnp.sum(f * f, axis=0, keepdims=True)


def _gn_apply(f, st_ref, gw_ref, gb_ref, gm_ref):
    inv_n = 1.0 / _NODES
    mean = st_ref[0:1, :] * inv_n
    mm = mean * gm_ref[...]
    var = st_ref[1:2, :] * inv_n - 2.0 * mm * mean + mm * mm
    o = f - mm
    g = gw_ref[...] * o * lax.rsqrt(var + 1e-5) + gb_ref[...]
    return 0.5 * g * (1.0 + lax.erf(g * 0.7071067811865476))


def _post1a_body(x_ref, agg_ref, dis_ref, b_ref, s_ref, f_ref, st_ref):
    i = pl.program_id(0)
    h = dis_ref[...] * agg_ref[...] + b_ref[...]
    x = x_ref[...]
    xn = jnp.sqrt(jnp.sum(x * x, axis=1, keepdims=True))
    f = _msg_norm(h, xn, s_ref[...]) + x
    f_ref[...] = f
    _accum_stats(i, st_ref, f)


_post1a = pl.pallas_call(
    _post1a_body,
    grid=(_NBLK,),
    in_specs=[
        pl.BlockSpec((_BLK, _D), lambda i: (i, 0)),
        pl.BlockSpec((_BLK, _D), lambda i: (i, 0)),
        pl.BlockSpec((_BLK, 1), lambda i: (i, 0)),
        pl.BlockSpec((1, _D), lambda i: (0, 0)),
        pl.BlockSpec((1, 1), lambda i: (0, 0)),
    ],
    out_specs=[
        pl.BlockSpec((_BLK, _D), lambda i: (i, 0)),
        pl.BlockSpec((8, _D), lambda i: (0, 0)),
    ],
    out_shape=[
        jax.ShapeDtypeStruct((_NODES, _D), _F32),
        jax.ShapeDtypeStruct((8, _D), _F32),
    ],
)


def _post1b_body(f_ref, st_ref, gw_ref, gb_ref, gm_ref, W_ref, dis_ref, w_ref,
                 fo_ref, msg_ref, hd_ref):
    g = _gn_apply(f_ref[...], st_ref, gw_ref, gb_ref, gm_ref)
    fo_ref[...] = g
    h = jnp.dot(g, W_ref[...], preferred_element_type=_F32)
    hd = h * dis_ref[...]
    hd_ref[...] = hd
    msg_ref[...] = hd * w_ref[...]


_post1b = pl.pallas_call(
    _post1b_body,
    grid=(_NBLK,),
    in_specs=[
        pl.BlockSpec((_BLK, _D), lambda i: (i, 0)),
        pl.BlockSpec((8, _D), lambda i: (0, 0)),
        pl.BlockSpec((1, _D), lambda i: (0, 0)),
        pl.BlockSpec((1, _D), lambda i: (0, 0)),
        pl.BlockSpec((1, _D), lambda i: (0, 0)),
        pl.BlockSpec((_D, _D), lambda i: (0, 0)),
        pl.BlockSpec((_BLK, 1), lambda i: (i, 0)),
        pl.BlockSpec((_BLK, 1), lambda i: (i, 0)),
    ],
    out_specs=[
        pl.BlockSpec((_BLK, _D), lambda i: (i, 0)),
        pl.BlockSpec((_BLK, _D), lambda i: (i, 0)),
        pl.BlockSpec((_BLK, _D), lambda i: (i, 0)),
    ],
    out_shape=[
        jax.ShapeDtypeStruct((_NODES, _D), _F32),
        jax.ShapeDtypeStruct((_NODES, _D), _F32),
        jax.ShapeDtypeStruct((_NODES, _D), _F32),
    ],
)


def _post2a_body(x_ref, f_ref, agg_ref, dis_ref, b_ref, s2_ref, s3_ref,
                 out_ref, st_ref):
    i = pl.program_id(0)
    h2 = dis_ref[...] * agg_ref[...] + b_ref[...]
    f = f_ref[...]
    fn = jnp.sqrt(jnp.sum(f * f, axis=1, keepdims=True))
    f2 = _msg_norm(h2, fn, s2_ref[...]) + f
    x = x_ref[...]
    xn = jnp.sqrt(jnp.sum(x * x, axis=1, keepdims=True))
    op = _msg_norm(f2, xn, s3_ref[...]) + x
    out_ref[...] = op
    _accum_stats(i, st_ref, op)


_post2a = pl.pallas_call(
    _post2a_body,
    grid=(_NBLK,),
    in_specs=[
        pl.BlockSpec((_BLK, _D), lambda i: (i, 0)),
        pl.BlockSpec((_BLK, _D), lambda i: (i, 0)),
        pl.BlockSpec((_BLK, _D), lambda i: (i, 0)),
        pl.BlockSpec((_BLK, 1), lambda i: (i, 0)),
        pl.BlockSpec((1, _D), lambda i: (0, 0)),
        pl.BlockSpec((1, 1), lambda i: (0, 0)),
        pl.BlockSpec((1, 1), lambda i: (0, 0)),
    ],
    out_specs=[
        pl.BlockSpec((_BLK, _D), lambda i: (i, 0)),
        pl.BlockSpec((8, _D), lambda i: (0, 0)),
    ],
    out_shape=[
        jax.ShapeDtypeStruct((_NODES, _D), _F32),
        jax.ShapeDtypeStruct((8, _D), _F32),
    ],
)


def _post2b_next_body(p_ref, st_ref, gw_ref, gb_ref, gm_ref, W_ref, dis_ref,
                      w_ref, out_ref, msg_ref, hd_ref):
    g = _gn_apply(p_ref[...], st_ref, gw_ref, gb_ref, gm_ref)
    out_ref[...] = g
    h = jnp.dot(g, W_ref[...], preferred_element_type=_F32)
    hd = h * dis_ref[...]
    hd_ref[...] = hd
    msg_ref[...] = hd * w_ref[...]


_post2b_next = pl.pallas_call(
    _post2b_next_body,
    grid=(_NBLK,),
    in_specs=[
        pl.BlockSpec((_BLK, _D), lambda i: (i, 0)),
        pl.BlockSpec((8, _D), lambda i: (0, 0)),
        pl.BlockSpec((1, _D), lambda i: (0, 0)),
        pl.BlockSpec((1, _D), lambda i: (0, 0)),
        pl.BlockSpec((1, _D), lambda i: (0, 0)),
        pl.BlockSpec((_D, _D), lambda i: (0, 0)),
        pl.BlockSpec((_BLK, 1), lambda i: (i, 0)),
        pl.BlockSpec((_BLK, 1), lambda i: (i, 0)),
    ],
    out_specs=[
        pl.BlockSpec((_BLK, _D), lambda i: (i, 0)),
        pl.BlockSpec((_BLK, _D), lambda i: (i, 0)),
        pl.BlockSpec((_BLK, _D), lambda i: (i, 0)),
    ],
    out_shape=[
        jax.ShapeDtypeStruct((_NODES, _D), _F32),
        jax.ShapeDtypeStruct((_NODES, _D), _F32),
        jax.ShapeDtypeStruct((_NODES, _D), _F32),
    ],
)


def _post2b_body(p_ref, st_ref, gw_ref, gb_ref, gm_ref, out_ref):
    out_ref[...] = _gn_apply(p_ref[...], st_ref, gw_ref, gb_ref, gm_ref)


_post2b = pl.pallas_call(
    _post2b_body,
    grid=(_NBLK,),
    in_specs=[
        pl.BlockSpec((_BLK, _D), lambda i: (i, 0)),
        pl.BlockSpec((8, _D), lambda i: (0, 0)),
        pl.BlockSpec((1, _D), lambda i: (0, 0)),
        pl.BlockSpec((1, _D), lambda i: (0, 0)),
        pl.BlockSpec((1, _D), lambda i: (0, 0)),
    ],
    out_specs=pl.BlockSpec((_BLK, _D), lambda i: (i, 0)),
    out_shape=jax.ShapeDtypeStruct((_NODES, _D), _F32),
)


@jax.jit
def kernel(data, W1, b1, s1, gn1_w, gn1_b, gn1_ms, W2, b2, s2, s3,
           gn3_w, gn3_b, gn3_ms):
    x = data.reshape(_NODES, _D)
    tl3, w3, dis3 = _edges(data)
    tl = tl3.reshape(_NODES // _IC, _IC)
    wcol = w3.reshape(_NODES, 1)
    discol = dis3.reshape(_NODES, 1)

    msg, hd = _pre(x, W1[0], discol, wcol)
    for l in range(2):
        agg = _scatter(msg, hd, tl)
        f_pre, st = _post1a(x, agg, discol, b1[l].reshape(1, _D),
                            s1[l].reshape(1, 1))
        f, msg, hd = _post1b(f_pre, st, gn1_w[l].reshape(1, _D),
                             gn1_b[l].reshape(1, _D), gn1_ms[l].reshape(1, _D),
                             W2[l], discol, wcol)
        agg2 = _scatter(msg, hd, tl)
        op, st2 = _post2a(x, f, agg2, discol, b2[l].reshape(1, _D),
                          s2[l].reshape(1, 1), s3[l].reshape(1, 1))
        if l == 0:
            x, msg, hd = _post2b_next(op, st2, gn3_w[l].reshape(1, _D),
                                      gn3_b[l].reshape(1, _D),
                                      gn3_ms[l].reshape(1, _D),
                                      W1[1], discol, wcol)
        else:
            x = _post2b(op, st2, gn3_w[l].reshape(1, _D),
                        gn3_b[l].reshape(1, _D), gn3_ms[l].reshape(1, _D))
    return x


# speedup vs baseline: 27.5652x; 1.0517x over previous
import functools

import jax
import jax.numpy as jnp
from jax import lax
from jax.experimental import pallas as pl
from jax.experimental.pallas import tpu as pltpu
from jax.experimental.pallas import tpu_sc as plsc

_B = 8
_N = 2048
_D = 128
_NODES = _B * _N
_THR = 0.7
_RB = 512
_NBLK = 16
_BLK = _NODES // _NBLK
_NC, _NS = 2, 16
_EPW = _NODES // (_NC * _NS)
_HALF = _NODES // _NC
_TRASH = _HALF
_IC = 128
_F32 = jnp.float32


def _edges_body(d_ref, W_ref, ts_ref, dis_ref, hd_ref):
    b = pl.program_id(0)
    X = d_ref[0]
    nchunks = _N // _RB
    ones_rb = jnp.ones((_RB, 1), _F32)

    def chunk(k, deg_acc):
        Xc = d_ref[0, pl.ds(k * _RB, _RB), :]
        S = lax.dot_general(Xc, X, (((1,), (1,)), ((), ())),
                            preferred_element_type=_F32)
        m = jnp.max(S, axis=1, keepdims=True)
        E = jnp.exp(S - m)
        r = 1.0 / jnp.sum(E, axis=1, keepdims=True)
        mask = E * r > _THR
        col = lax.broadcasted_iota(jnp.int32, (_RB, _N), 1)
        am = jnp.min(jnp.where(mask, col, _N), axis=1)
        ts = jnp.where(am < _N, am + (b % (_B // _NC)) * _N, _TRASH)
        ts_ref[0, 0, pl.ds(k * _RB, _RB)] = ts
        degp = lax.dot_general(mask.astype(_F32), ones_rb,
                               (((0,), (0,)), ((), ())),
                               preferred_element_type=_F32)
        return deg_acc + degp

    deg = lax.fori_loop(0, nchunks, chunk, jnp.zeros((_N, 1), _F32))
    dis = lax.rsqrt(1.0 + deg)
    dis_ref[...] = dis
    h = jnp.dot(X, W_ref[...], preferred_element_type=_F32)
    hd_ref[...] = h * dis


_edges = pl.pallas_call(
    _edges_body,
    grid=(_B,),
    in_specs=[
        pl.BlockSpec((1, _N, _D), lambda b: (b, 0, 0)),
        pl.BlockSpec((_D, _D), lambda b: (0, 0)),
    ],
    out_specs=[
        pl.BlockSpec((1, 1, _N), lambda b: (b, 0, 0)),
        pl.BlockSpec((_N, 1), lambda b: (b, 0)),
        pl.BlockSpec((_N, _D), lambda b: (b, 0)),
    ],
    out_shape=[
        jax.ShapeDtypeStruct((_B, 1, _N), jnp.int32),
        jax.ShapeDtypeStruct((_NODES, 1), _F32),
        jax.ShapeDtypeStruct((_NODES, _D), _F32),
    ],
)


def _scatter_body(hd_hbm, ts_hbm, out_hbm, shared, buf, idx_v):
    c = lax.axis_index("c")
    s = lax.axis_index("s")
    base = c * _HALF + s * _EPW
    pltpu.sync_copy(hd_hbm.at[pl.ds(base, _EPW)],
                    shared.at[pl.ds(s * _EPW, _EPW)])
    wid = c * _NS + s
    pltpu.sync_copy(ts_hbm.at[pl.ds(wid * (_EPW // _IC), _EPW // _IC)], idx_v)
    plsc.subcore_barrier()
    for j in range(_EPW // _IC):
        pltpu.sync_copy(hd_hbm.at[pl.ds(base + j * _IC, _IC)], buf)
        pltpu.sync_copy(buf, shared.at[idx_v.at[j]], add=True)
    plsc.subcore_barrier()
    pltpu.sync_copy(shared.at[pl.ds(s * _EPW, _EPW)],
                    out_hbm.at[pl.ds(base, _EPW)])


@functools.cache
def _get_scatter():
    return pl.kernel(
        _scatter_body,
        out_type=jax.ShapeDtypeStruct((_NODES, _D), _F32),
        mesh=plsc.VectorSubcoreMesh(core_axis_name="c", subcore_axis_name="s",
                                    num_cores=_NC, num_subcores=_NS),
        scratch_types=[
            pltpu.VMEM_SHARED((_HALF + 16, _D), _F32),
            pltpu.VMEM((_IC, _D), _F32),
            pltpu.VMEM((_EPW // _IC, _IC), jnp.int32),
        ],
    )


def _scatter(hd, ts):
    return _get_scatter()(hd, ts)


def _msg_norm(h, x_norm, s):
    hn = jnp.sqrt(jnp.sum(h * h, axis=1, keepdims=True))
    return h / jnp.maximum(hn, 1e-12) * x_norm * s


def _accum_stats(i, st_ref, f):
    @pl.when(i == 0)
    def _():
        st_ref[...] = jnp.zeros_like(st_ref)

    st_ref[0:1, :] += jnp.sum(f, axis=0, keepdims=True)
    st_ref[1:2, :] += jnp.sum(f * f, axis=0, keepdims=True)


def _gn_apply(f, st_ref, gw_ref, gb_ref, gm_ref):
    inv_n = 1.0 / _NODES
    mean = st_ref[0:1, :] * inv_n
    mm = mean * gm_ref[...]
    var = st_ref[1:2, :] * inv_n - 2.0 * mm * mean + mm * mm
    o = f - mm
    g = gw_ref[...] * o * lax.rsqrt(var + 1e-5) + gb_ref[...]
    return 0.5 * g * (1.0 + lax.erf(g * 0.7071067811865476))


def _post1a_body(x_ref, agg_ref, dis_ref, b_ref, s_ref, f_ref, st_ref):
    i = pl.program_id(0)
    h = dis_ref[...] * agg_ref[...] + b_ref[...]
    x = x_ref[...]
    xn = jnp.sqrt(jnp.sum(x * x, axis=1, keepdims=True))
    f = _msg_norm(h, xn, s_ref[...]) + x
    f_ref[...] = f
    _accum_stats(i, st_ref, f)


_post1a = pl.pallas_call(
    _post1a_body,
    grid=(_NBLK,),
    in_specs=[
        pl.BlockSpec((_BLK, _D), lambda i: (i, 0)),
        pl.BlockSpec((_BLK, _D), lambda i: (i, 0)),
        pl.BlockSpec((_BLK, 1), lambda i: (i, 0)),
        pl.BlockSpec((1, _D), lambda i: (0, 0)),
        pl.BlockSpec((1, 1), lambda i: (0, 0)),
    ],
    out_specs=[
        pl.BlockSpec((_BLK, _D), lambda i: (i, 0)),
        pl.BlockSpec((8, _D), lambda i: (0, 0)),
    ],
    out_shape=[
        jax.ShapeDtypeStruct((_NODES, _D), _F32),
        jax.ShapeDtypeStruct((8, _D), _F32),
    ],
)


def _post1b_body(f_ref, st_ref, gw_ref, gb_ref, gm_ref, W_ref, dis_ref,
                 fo_ref, hd_ref):
    g = _gn_apply(f_ref[...], st_ref, gw_ref, gb_ref, gm_ref)
    fo_ref[...] = g
    h = jnp.dot(g, W_ref[...], preferred_element_type=_F32)
    hd_ref[...] = h * dis_ref[...]


_post1b = pl.pallas_call(
    _post1b_body,
    grid=(_NBLK,),
    in_specs=[
        pl.BlockSpec((_BLK, _D), lambda i: (i, 0)),
        pl.BlockSpec((8, _D), lambda i: (0, 0)),
        pl.BlockSpec((1, _D), lambda i: (0, 0)),
        pl.BlockSpec((1, _D), lambda i: (0, 0)),
        pl.BlockSpec((1, _D), lambda i: (0, 0)),
        pl.BlockSpec((_D, _D), lambda i: (0, 0)),
        pl.BlockSpec((_BLK, 1), lambda i: (i, 0)),
    ],
    out_specs=[
        pl.BlockSpec((_BLK, _D), lambda i: (i, 0)),
        pl.BlockSpec((_BLK, _D), lambda i: (i, 0)),
    ],
    out_shape=[
        jax.ShapeDtypeStruct((_NODES, _D), _F32),
        jax.ShapeDtypeStruct((_NODES, _D), _F32),
    ],
)


def _post2a_body(x_ref, f_ref, agg_ref, dis_ref, b_ref, s2_ref, s3_ref,
                 out_ref, st_ref):
    i = pl.program_id(0)
    h2 = dis_ref[...] * agg_ref[...] + b_ref[...]
    f = f_ref[...]
    fn = jnp.sqrt(jnp.sum(f * f, axis=1, keepdims=True))
    f2 = _msg_norm(h2, fn, s2_ref[...]) + f
    x = x_ref[...]
    xn = jnp.sqrt(jnp.sum(x * x, axis=1, keepdims=True))
    op = _msg_norm(f2, xn, s3_ref[...]) + x
    out_ref[...] = op
    _accum_stats(i, st_ref, op)


_post2a = pl.pallas_call(
    _post2a_body,
    grid=(_NBLK,),
    in_specs=[
        pl.BlockSpec((_BLK, _D), lambda i: (i, 0)),
        pl.BlockSpec((_BLK, _D), lambda i: (i, 0)),
        pl.BlockSpec((_BLK, _D), lambda i: (i, 0)),
        pl.BlockSpec((_BLK, 1), lambda i: (i, 0)),
        pl.BlockSpec((1, _D), lambda i: (0, 0)),
        pl.BlockSpec((1, 1), lambda i: (0, 0)),
        pl.BlockSpec((1, 1), lambda i: (0, 0)),
    ],
    out_specs=[
        pl.BlockSpec((_BLK, _D), lambda i: (i, 0)),
        pl.BlockSpec((8, _D), lambda i: (0, 0)),
    ],
    out_shape=[
        jax.ShapeDtypeStruct((_NODES, _D), _F32),
        jax.ShapeDtypeStruct((8, _D), _F32),
    ],
)


def _post2b_next_body(p_ref, st_ref, gw_ref, gb_ref, gm_ref, W_ref, dis_ref,
                      out_ref, hd_ref):
    g = _gn_apply(p_ref[...], st_ref, gw_ref, gb_ref, gm_ref)
    out_ref[...] = g
    h = jnp.dot(g, W_ref[...], preferred_element_type=_F32)
    hd_ref[...] = h * dis_ref[...]


_post2b_next = pl.pallas_call(
    _post2b_next_body,
    grid=(_NBLK,),
    in_specs=[
        pl.BlockSpec((_BLK, _D), lambda i: (i, 0)),
        pl.BlockSpec((8, _D), lambda i: (0, 0)),
        pl.BlockSpec((1, _D), lambda i: (0, 0)),
        pl.BlockSpec((1, _D), lambda i: (0, 0)),
        pl.BlockSpec((1, _D), lambda i: (0, 0)),
        pl.BlockSpec((_D, _D), lambda i: (0, 0)),
        pl.BlockSpec((_BLK, 1), lambda i: (i, 0)),
    ],
    out_specs=[
        pl.BlockSpec((_BLK, _D), lambda i: (i, 0)),
        pl.BlockSpec((_BLK, _D), lambda i: (i, 0)),
    ],
    out_shape=[
        jax.ShapeDtypeStruct((_NODES, _D), _F32),
        jax.ShapeDtypeStruct((_NODES, _D), _F32),
    ],
)


def _post2b_body(p_ref, st_ref, gw_ref, gb_ref, gm_ref, out_ref):
    out_ref[...] = _gn_apply(p_ref[...], st_ref, gw_ref, gb_ref, gm_ref)


_post2b = pl.pallas_call(
    _post2b_body,
    grid=(_NBLK,),
    in_specs=[
        pl.BlockSpec((_BLK, _D), lambda i: (i, 0)),
        pl.BlockSpec((8, _D), lambda i: (0, 0)),
        pl.BlockSpec((1, _D), lambda i: (0, 0)),
        pl.BlockSpec((1, _D), lambda i: (0, 0)),
        pl.BlockSpec((1, _D), lambda i: (0, 0)),
    ],
    out_specs=pl.BlockSpec((_BLK, _D), lambda i: (i, 0)),
    out_shape=jax.ShapeDtypeStruct((_NODES, _D), _F32),
)


@jax.jit
def kernel(data, W1, b1, s1, gn1_w, gn1_b, gn1_ms, W2, b2, s2, s3,
           gn3_w, gn3_b, gn3_ms):
    x = data.reshape(_NODES, _D)
    ts3, dis, hd = _edges(data, W1[0])
    ts = ts3.reshape(_NODES // _IC, _IC)

    for l in range(2):
        agg = _scatter(hd, ts)
        f_pre, st = _post1a(x, agg, dis, b1[l].reshape(1, _D),
                            s1[l].reshape(1, 1))
        f, hd = _post1b(f_pre, st, gn1_w[l].reshape(1, _D),
                        gn1_b[l].reshape(1, _D), gn1_ms[l].reshape(1, _D),
                        W2[l], dis)
        agg2 = _scatter(hd, ts)
        op, st2 = _post2a(x, f, agg2, dis, b2[l].reshape(1, _D),
                          s2[l].reshape(1, 1), s3[l].reshape(1, 1))
        if l == 0:
            x, hd = _post2b_next(op, st2, gn3_w[l].reshape(1, _D),
                                 gn3_b[l].reshape(1, _D),
                                 gn3_ms[l].reshape(1, _D), W1[1], dis)
        else:
            x = _post2b(op, st2, gn3_w[l].reshape(1, _D),
                        gn3_b[l].reshape(1, _D), gn3_ms[l].reshape(1, _D))
    return x


# speedup vs baseline: 28.0612x; 1.0180x over previous
import functools

import jax
import jax.numpy as jnp
from jax import lax
from jax.experimental import pallas as pl
from jax.experimental.pallas import tpu as pltpu
from jax.experimental.pallas import tpu_sc as plsc

_B = 8
_N = 2048
_D = 128
_NODES = _B * _N
_THR = 0.7
_RB = 512
_NBLK = 16
_BLK = _NODES // _NBLK
_NC, _NS = 2, 16
_EPW = _NODES // (_NC * _NS)
_HALF = _NODES // _NC
_TRASH = _HALF
_IC = 128
_F32 = jnp.float32


def _edges_body(d_ref, W_ref, ts_ref, dis_ref, hd_ref):
    b = pl.program_id(0)
    X = d_ref[0]
    nchunks = _N // _RB
    ones_rb = jnp.ones((_RB, 1), _F32)
    ones_n = jnp.ones((_N, 1), _F32)
    iota_n = lax.broadcasted_iota(jnp.int32, (_N, 1), 0).astype(_F32)
    off = (b % (_B // _NC)) * _N

    def chunk(k, deg_acc):
        Xc = d_ref[0, pl.ds(k * _RB, _RB), :]
        S = lax.dot_general(Xc, X, (((1,), (1,)), ((), ())),
                            preferred_element_type=_F32)
        m = jnp.max(S, axis=1, keepdims=True)
        E = jnp.exp(S - m)
        sm = jnp.sum(E, axis=1, keepdims=True)
        mask = E > _THR * sm
        maskf = mask.astype(_F32)
        col = lax.broadcasted_iota(jnp.int32, (_RB, _N), 1)
        am = jnp.min(jnp.where(mask, col, _N), axis=1)
        ts = jnp.where(am < _N, am + off, _TRASH)
        ts_ref[0, 0, pl.ds(k * _RB, _RB)] = ts
        degp = lax.dot_general(maskf, ones_rb,
                               (((0,), (0,)), ((), ())),
                               preferred_element_type=_F32)
        return deg_acc + degp

    deg = lax.fori_loop(0, nchunks, chunk, jnp.zeros((_N, 1), _F32))
    dis = lax.rsqrt(1.0 + deg)
    dis_ref[...] = dis
    h = jnp.dot(X, W_ref[...], preferred_element_type=_F32)
    hd_ref[...] = h * dis


_edges = pl.pallas_call(
    _edges_body,
    grid=(_B,),
    in_specs=[
        pl.BlockSpec((1, _N, _D), lambda b: (b, 0, 0)),
        pl.BlockSpec((_D, _D), lambda b: (0, 0)),
    ],
    out_specs=[
        pl.BlockSpec((1, 1, _N), lambda b: (b, 0, 0)),
        pl.BlockSpec((_N, 1), lambda b: (b, 0)),
        pl.BlockSpec((_N, _D), lambda b: (b, 0)),
    ],
    out_shape=[
        jax.ShapeDtypeStruct((_B, 1, _N), jnp.int32),
        jax.ShapeDtypeStruct((_NODES, 1), _F32),
        jax.ShapeDtypeStruct((_NODES, _D), _F32),
    ],
)


def _scatter_body(hd_hbm, ts_hbm, out_hbm, shared, buf0, buf1, idx_v,
                  sem_i, sem0, sem1):
    c = lax.axis_index("c")
    s = lax.axis_index("s")
    base = c * _HALF + s * _EPW
    nch = _EPW // _IC
    bufs = (buf0, buf1)
    sems = (sem0, sem1)
    init = pltpu.async_copy(hd_hbm.at[pl.ds(base, _EPW)],
                            shared.at[pl.ds(s * _EPW, _EPW)], sem_i)
    wid = c * _NS + s
    pltpu.sync_copy(ts_hbm.at[pl.ds(wid * nch, nch)], idx_v)
    init.wait()
    plsc.subcore_barrier()
    for j in range(nch):
        pltpu.sync_copy(hd_hbm.at[pl.ds(base + j * _IC, _IC)], buf0)
        pltpu.sync_copy(buf0, shared.at[idx_v.at[j]], add=True)
    plsc.subcore_barrier()
    pltpu.sync_copy(shared.at[pl.ds(s * _EPW, _EPW)],
                    out_hbm.at[pl.ds(base, _EPW)])


@functools.cache
def _get_scatter():
    return pl.kernel(
        _scatter_body,
        out_type=jax.ShapeDtypeStruct((_NODES, _D), _F32),
        mesh=plsc.VectorSubcoreMesh(core_axis_name="c", subcore_axis_name="s",
                                    num_cores=_NC, num_subcores=_NS),
        scratch_types=[
            pltpu.VMEM_SHARED((_HALF + 16, _D), _F32),
            pltpu.VMEM((_IC, _D), _F32),
            pltpu.VMEM((_IC, _D), _F32),
            pltpu.VMEM((_EPW // _IC, _IC), jnp.int32),
            pltpu.SemaphoreType.DMA,
            pltpu.SemaphoreType.DMA,
            pltpu.SemaphoreType.DMA,
        ],
    )


def _scatter(hd, ts):
    return _get_scatter()(hd, ts)


def _msg_norm(h, x_norm, s):
    hn = jnp.sqrt(jnp.sum(h * h, axis=1, keepdims=True))
    return h / jnp.maximum(hn, 1e-12) * x_norm * s


def _accum_stats(i, st_ref, f):
    @pl.when(i == 0)
    def _():
        st_ref[...] = jnp.zeros_like(st_ref)

    st_ref[0:1, :] += jnp.sum(f, axis=0, keepdims=True)
    st_ref[1:2, :] += jnp.sum(f * f, axis=0, keepdims=True)


def _gn_apply(f, st_ref, gw_ref, gb_ref, gm_ref):
    inv_n = 1.0 / _NODES
    mean = st_ref[0:1, :] * inv_n
    mm = mean * gm_ref[...]
    var = st_ref[1:2, :] * inv_n - 2.0 * mm * mean + mm * mm
    o = f - mm
    g = gw_ref[...] * o * lax.rsqrt(var + 1e-5) + gb_ref[...]
    return 0.5 * g * (1.0 + lax.erf(g * 0.7071067811865476))


def _post1a_body(x_ref, agg_ref, dis_ref, b_ref, s_ref, f_ref, st_ref):
    i = pl.program_id(0)
    h = dis_ref[...] * agg_ref[...] + b_ref[...]
    x = x_ref[...]
    xn = jnp.sqrt(jnp.sum(x * x, axis=1, keepdims=True))
    f = _msg_norm(h, xn, s_ref[...]) + x
    f_ref[...] = f
    _accum_stats(i, st_ref, f)


_post1a = pl.pallas_call(
    _post1a_body,
    grid=(_NBLK,),
    in_specs=[
        pl.BlockSpec((_BLK, _D), lambda i: (i, 0)),
        pl.BlockSpec((_BLK, _D), lambda i: (i, 0)),
        pl.BlockSpec((_BLK, 1), lambda i: (i, 0)),
        pl.BlockSpec((1, _D), lambda i: (0, 0)),
        pl.BlockSpec((1, 1), lambda i: (0, 0)),
    ],
    out_specs=[
        pl.BlockSpec((_BLK, _D), lambda i: (i, 0)),
        pl.BlockSpec((8, _D), lambda i: (0, 0)),
    ],
    out_shape=[
        jax.ShapeDtypeStruct((_NODES, _D), _F32),
        jax.ShapeDtypeStruct((8, _D), _F32),
    ],
)


def _post1b_body(f_ref, st_ref, gw_ref, gb_ref, gm_ref, W_ref, dis_ref,
                 fo_ref, hd_ref):
    g = _gn_apply(f_ref[...], st_ref, gw_ref, gb_ref, gm_ref)
    fo_ref[...] = g
    h = jnp.dot(g, W_ref[...], preferred_element_type=_F32)
    hd_ref[...] = h * dis_ref[...]


_post1b = pl.pallas_call(
    _post1b_body,
    grid=(_NBLK,),
    in_specs=[
        pl.BlockSpec((_BLK, _D), lambda i: (i, 0)),
        pl.BlockSpec((8, _D), lambda i: (0, 0)),
        pl.BlockSpec((1, _D), lambda i: (0, 0)),
        pl.BlockSpec((1, _D), lambda i: (0, 0)),
        pl.BlockSpec((1, _D), lambda i: (0, 0)),
        pl.BlockSpec((_D, _D), lambda i: (0, 0)),
        pl.BlockSpec((_BLK, 1), lambda i: (i, 0)),
    ],
    out_specs=[
        pl.BlockSpec((_BLK, _D), lambda i: (i, 0)),
        pl.BlockSpec((_BLK, _D), lambda i: (i, 0)),
    ],
    out_shape=[
        jax.ShapeDtypeStruct((_NODES, _D), _F32),
        jax.ShapeDtypeStruct((_NODES, _D), _F32),
    ],
)


def _post2a_body(x_ref, f_ref, agg_ref, dis_ref, b_ref, s2_ref, s3_ref,
                 out_ref, st_ref):
    i = pl.program_id(0)
    h2 = dis_ref[...] * agg_ref[...] + b_ref[...]
    f = f_ref[...]
    fn = jnp.sqrt(jnp.sum(f * f, axis=1, keepdims=True))
    f2 = _msg_norm(h2, fn, s2_ref[...]) + f
    x = x_ref[...]
    xn = jnp.sqrt(jnp.sum(x * x, axis=1, keepdims=True))
    op = _msg_norm(f2, xn, s3_ref[...]) + x
    out_ref[...] = op
    _accum_stats(i, st_ref, op)


_post2a = pl.pallas_call(
    _post2a_body,
    grid=(_NBLK,),
    in_specs=[
        pl.BlockSpec((_BLK, _D), lambda i: (i, 0)),
        pl.BlockSpec((_BLK, _D), lambda i: (i, 0)),
        pl.BlockSpec((_BLK, _D), lambda i: (i, 0)),
        pl.BlockSpec((_BLK, 1), lambda i: (i, 0)),
        pl.BlockSpec((1, _D), lambda i: (0, 0)),
        pl.BlockSpec((1, 1), lambda i: (0, 0)),
        pl.BlockSpec((1, 1), lambda i: (0, 0)),
    ],
    out_specs=[
        pl.BlockSpec((_BLK, _D), lambda i: (i, 0)),
        pl.BlockSpec((8, _D), lambda i: (0, 0)),
    ],
    out_shape=[
        jax.ShapeDtypeStruct((_NODES, _D), _F32),
        jax.ShapeDtypeStruct((8, _D), _F32),
    ],
)


def _post2b_next_body(p_ref, st_ref, gw_ref, gb_ref, gm_ref, W_ref, dis_ref,
                      out_ref, hd_ref):
    g = _gn_apply(p_ref[...], st_ref, gw_ref, gb_ref, gm_ref)
    out_ref[...] = g
    h = jnp.dot(g, W_ref[...], preferred_element_type=_F32)
    hd_ref[...] = h * dis_ref[...]


_post2b_next = pl.pallas_call(
    _post2b_next_body,
    grid=(_NBLK,),
    in_specs=[
        pl.BlockSpec((_BLK, _D), lambda i: (i, 0)),
        pl.BlockSpec((8, _D), lambda i: (0, 0)),
        pl.BlockSpec((1, _D), lambda i: (0, 0)),
        pl.BlockSpec((1, _D), lambda i: (0, 0)),
        pl.BlockSpec((1, _D), lambda i: (0, 0)),
        pl.BlockSpec((_D, _D), lambda i: (0, 0)),
        pl.BlockSpec((_BLK, 1), lambda i: (i, 0)),
    ],
    out_specs=[
        pl.BlockSpec((_BLK, _D), lambda i: (i, 0)),
        pl.BlockSpec((_BLK, _D), lambda i: (i, 0)),
    ],
    out_shape=[
        jax.ShapeDtypeStruct((_NODES, _D), _F32),
        jax.ShapeDtypeStruct((_NODES, _D), _F32),
    ],
)


def _post2b_body(p_ref, st_ref, gw_ref, gb_ref, gm_ref, out_ref):
    out_ref[...] = _gn_apply(p_ref[...], st_ref, gw_ref, gb_ref, gm_ref)


_post2b = pl.pallas_call(
    _post2b_body,
    grid=(_NBLK,),
    in_specs=[
        pl.BlockSpec((_BLK, _D), lambda i: (i, 0)),
        pl.BlockSpec((8, _D), lambda i: (0, 0)),
        pl.BlockSpec((1, _D), lambda i: (0, 0)),
        pl.BlockSpec((1, _D), lambda i: (0, 0)),
        pl.BlockSpec((1, _D), lambda i: (0, 0)),
    ],
    out_specs=pl.BlockSpec((_BLK, _D), lambda i: (i, 0)),
    out_shape=jax.ShapeDtypeStruct((_NODES, _D), _F32),
)


@jax.jit
def kernel(data, W1, b1, s1, gn1_w, gn1_b, gn1_ms, W2, b2, s2, s3,
           gn3_w, gn3_b, gn3_ms):
    x = data.reshape(_NODES, _D)
    tsc, dis, hd = _edges(data, W1[0])
    ts = tsc.reshape(_NODES // _IC, _IC)

    for l in range(2):
        agg = _scatter(hd, ts)
        f_pre, st = _post1a(x, agg, dis, b1[l].reshape(1, _D),
                            s1[l].reshape(1, 1))
        f, hd = _post1b(f_pre, st, gn1_w[l].reshape(1, _D),
                        gn1_b[l].reshape(1, _D), gn1_ms[l].reshape(1, _D),
                        W2[l], dis)
        agg2 = _scatter(hd, ts)
        op, st2 = _post2a(x, f, agg2, dis, b2[l].reshape(1, _D),
                          s2[l].reshape(1, 1), s3[l].reshape(1, 1))
        if l == 0:
            x, hd = _post2b_next(op, st2, gn3_w[l].reshape(1, _D),
                                 gn3_b[l].reshape(1, _D),
                                 gn3_ms[l].reshape(1, _D), W1[1], dis)
        else:
            x = _post2b(op, st2, gn3_w[l].reshape(1, _D),
                        gn3_b[l].reshape(1, _D), gn3_ms[l].reshape(1, _D))
    return x


# speedup vs baseline: 29.2972x; 1.0440x over previous
import functools

import jax
import jax.numpy as jnp
from jax import lax
from jax.experimental import pallas as pl
from jax.experimental.pallas import tpu as pltpu
from jax.experimental.pallas import tpu_sc as plsc

_B = 8
_N = 2048
_D = 128
_NODES = _B * _N
_THR = 0.7
_RB = 512
_NBLK = 16
_BLK = _NODES // _NBLK
_NC, _NS = 2, 16
_EPW = _NODES // (_NC * _NS)
_HALF = _NODES // _NC
_TRASH = _HALF
_IC = 128
_F32 = jnp.float32


def _edges_body(d_ref, W_ref, ts_ref, dis_ref, hd_ref):
    b = pl.program_id(0)
    X = d_ref[0]
    nchunks = _N // _RB
    ones_rb = jnp.ones((_RB, 1), _F32)
    ones_n = jnp.ones((_N, 1), _F32)
    iota_n = lax.broadcasted_iota(jnp.int32, (_N, 1), 0).astype(_F32)
    off = (b % (_B // _NC)) * _N

    def chunk(k, deg_acc):
        Xc = d_ref[0, pl.ds(k * _RB, _RB), :]
        S = lax.dot_general(Xc, X, (((1,), (1,)), ((), ())),
                            preferred_element_type=_F32)
        m = jnp.max(S, axis=1, keepdims=True)
        E = jnp.exp(S - m)
        sm = jnp.sum(E, axis=1, keepdims=True)
        mask = E > _THR * sm
        maskf = mask.astype(_F32)
        col = lax.broadcasted_iota(jnp.int32, (_RB, _N), 1)
        am = jnp.min(jnp.where(mask, col, _N), axis=1)
        ts = jnp.where(am < _N, am + off, _TRASH)
        ts_ref[0, 0, pl.ds(k * _RB, _RB)] = ts
        degp = lax.dot_general(maskf, ones_rb,
                               (((0,), (0,)), ((), ())),
                               preferred_element_type=_F32)
        return deg_acc + degp

    deg = lax.fori_loop(0, nchunks, chunk, jnp.zeros((_N, 1), _F32))
    dis = lax.rsqrt(1.0 + deg)
    dis_ref[...] = dis
    h = jnp.dot(X, W_ref[...], preferred_element_type=_F32)
    hd_ref[...] = h * dis


_edges = pl.pallas_call(
    _edges_body,
    grid=(_B,),
    in_specs=[
        pl.BlockSpec((1, _N, _D), lambda b: (b, 0, 0)),
        pl.BlockSpec((_D, _D), lambda b: (0, 0)),
    ],
    out_specs=[
        pl.BlockSpec((1, 1, _N), lambda b: (b, 0, 0)),
        pl.BlockSpec((_N, 1), lambda b: (b, 0)),
        pl.BlockSpec((_N, _D), lambda b: (b, 0)),
    ],
    out_shape=[
        jax.ShapeDtypeStruct((_B, 1, _N), jnp.int32),
        jax.ShapeDtypeStruct((_NODES, 1), _F32),
        jax.ShapeDtypeStruct((_NODES, _D), _F32),
    ],
)


def _scatter_body(hd_hbm, ts_hbm, out_hbm, shared, buf0, buf1, idx_v,
                  sem_i, sem0, sem1):
    c = lax.axis_index("c")
    s = lax.axis_index("s")
    base = c * _HALF + s * _EPW
    nch = _EPW // _IC
    bufs = (buf0, buf1)
    sems = (sem0, sem1)
    init = pltpu.async_copy(hd_hbm.at[pl.ds(base, _EPW)],
                            shared.at[pl.ds(s * _EPW, _EPW)], sem_i)
    wid = c * _NS + s
    pltpu.sync_copy(ts_hbm.at[pl.ds(wid * nch, nch)], idx_v)
    gathers = [pltpu.async_copy(hd_hbm.at[pl.ds(base, _IC)], buf0, sem0)]
    init.wait()
    plsc.subcore_barrier()
    for j in range(nch):
        gathers[j].wait()
        if j + 1 < nch:
            gathers.append(
                pltpu.async_copy(hd_hbm.at[pl.ds(base + (j + 1) * _IC, _IC)],
                                 bufs[(j + 1) % 2], sems[(j + 1) % 2]))
        pltpu.sync_copy(bufs[j % 2], shared.at[idx_v.at[j]], add=True)
    plsc.subcore_barrier()
    pltpu.sync_copy(shared.at[pl.ds(s * _EPW, _EPW)],
                    out_hbm.at[pl.ds(base, _EPW)])


@functools.cache
def _get_scatter():
    return pl.kernel(
        _scatter_body,
        out_type=jax.ShapeDtypeStruct((_NODES, _D), _F32),
        mesh=plsc.VectorSubcoreMesh(core_axis_name="c", subcore_axis_name="s",
                                    num_cores=_NC, num_subcores=_NS),
        scratch_types=[
            pltpu.VMEM_SHARED((_HALF + 16, _D), _F32),
            pltpu.VMEM((_IC, _D), _F32),
            pltpu.VMEM((_IC, _D), _F32),
            pltpu.VMEM((_EPW // _IC, _IC), jnp.int32),
            pltpu.SemaphoreType.DMA,
            pltpu.SemaphoreType.DMA,
            pltpu.SemaphoreType.DMA,
        ],
    )


def _scatter(hd, ts):
    return _get_scatter()(hd, ts)


def _msg_norm(h, x_norm, s):
    hn = jnp.sqrt(jnp.sum(h * h, axis=1, keepdims=True))
    return h / jnp.maximum(hn, 1e-12) * x_norm * s


def _accum_stats(i, st_ref, f):
    @pl.when(i == 0)
    def _():
        st_ref[...] = jnp.zeros_like(st_ref)

    st_ref[0:1, :] += jnp.sum(f, axis=0, keepdims=True)
    st_ref[1:2, :] += jnp.sum(f * f, axis=0, keepdims=True)


def _gn_apply(f, st_ref, gw_ref, gb_ref, gm_ref):
    inv_n = 1.0 / _NODES
    mean = st_ref[0:1, :] * inv_n
    mm = mean * gm_ref[...]
    var = st_ref[1:2, :] * inv_n - 2.0 * mm * mean + mm * mm
    o = f - mm
    g = gw_ref[...] * o * lax.rsqrt(var + 1e-5) + gb_ref[...]
    return 0.5 * g * (1.0 + lax.erf(g * 0.7071067811865476))


def _post1a_body(x_ref, agg_ref, dis_ref, b_ref, s_ref, f_ref, st_ref):
    i = pl.program_id(0)
    h = dis_ref[...] * agg_ref[...] + b_ref[...]
    x = x_ref[...]
    xn = jnp.sqrt(jnp.sum(x * x, axis=1, keepdims=True))
    f = _msg_norm(h, xn, s_ref[...]) + x
    f_ref[...] = f
    _accum_stats(i, st_ref, f)


_post1a = pl.pallas_call(
    _post1a_body,
    grid=(_NBLK,),
    in_specs=[
        pl.BlockSpec((_BLK, _D), lambda i: (i, 0)),
        pl.BlockSpec((_BLK, _D), lambda i: (i, 0)),
        pl.BlockSpec((_BLK, 1), lambda i: (i, 0)),
        pl.BlockSpec((1, _D), lambda i: (0, 0)),
        pl.BlockSpec((1, 1), lambda i: (0, 0)),
    ],
    out_specs=[
        pl.BlockSpec((_BLK, _D), lambda i: (i, 0)),
        pl.BlockSpec((8, _D), lambda i: (0, 0)),
    ],
    out_shape=[
        jax.ShapeDtypeStruct((_NODES, _D), _F32),
        jax.ShapeDtypeStruct((8, _D), _F32),
    ],
)


def _post1b_body(f_ref, st_ref, gw_ref, gb_ref, gm_ref, W_ref, dis_ref,
                 hd_ref):
    g = _gn_apply(f_ref[...], st_ref, gw_ref, gb_ref, gm_ref)
    h = jnp.dot(g, W_ref[...], preferred_element_type=_F32)
    hd_ref[...] = h * dis_ref[...]


_post1b = pl.pallas_call(
    _post1b_body,
    grid=(_NBLK,),
    in_specs=[
        pl.BlockSpec((_BLK, _D), lambda i: (i, 0)),
        pl.BlockSpec((8, _D), lambda i: (0, 0)),
        pl.BlockSpec((1, _D), lambda i: (0, 0)),
        pl.BlockSpec((1, _D), lambda i: (0, 0)),
        pl.BlockSpec((1, _D), lambda i: (0, 0)),
        pl.BlockSpec((_D, _D), lambda i: (0, 0)),
        pl.BlockSpec((_BLK, 1), lambda i: (i, 0)),
    ],
    out_specs=pl.BlockSpec((_BLK, _D), lambda i: (i, 0)),
    out_shape=jax.ShapeDtypeStruct((_NODES, _D), _F32),
)


def _post2a_body(x_ref, fp_ref, st1_ref, gw_ref, gb_ref, gm_ref, agg_ref,
                 dis_ref, b_ref, s2_ref, s3_ref, out_ref, st_ref):
    i = pl.program_id(0)
    f = _gn_apply(fp_ref[...], st1_ref, gw_ref, gb_ref, gm_ref)
    h2 = dis_ref[...] * agg_ref[...] + b_ref[...]
    fn = jnp.sqrt(jnp.sum(f * f, axis=1, keepdims=True))
    f2 = _msg_norm(h2, fn, s2_ref[...]) + f
    x = x_ref[...]
    xn = jnp.sqrt(jnp.sum(x * x, axis=1, keepdims=True))
    op = _msg_norm(f2, xn, s3_ref[...]) + x
    out_ref[...] = op
    _accum_stats(i, st_ref, op)


_post2a = pl.pallas_call(
    _post2a_body,
    grid=(_NBLK,),
    in_specs=[
        pl.BlockSpec((_BLK, _D), lambda i: (i, 0)),
        pl.BlockSpec((_BLK, _D), lambda i: (i, 0)),
        pl.BlockSpec((8, _D), lambda i: (0, 0)),
        pl.BlockSpec((1, _D), lambda i: (0, 0)),
        pl.BlockSpec((1, _D), lambda i: (0, 0)),
        pl.BlockSpec((1, _D), lambda i: (0, 0)),
        pl.BlockSpec((_BLK, _D), lambda i: (i, 0)),
        pl.BlockSpec((_BLK, 1), lambda i: (i, 0)),
        pl.BlockSpec((1, _D), lambda i: (0, 0)),
        pl.BlockSpec((1, 1), lambda i: (0, 0)),
        pl.BlockSpec((1, 1), lambda i: (0, 0)),
    ],
    out_specs=[
        pl.BlockSpec((_BLK, _D), lambda i: (i, 0)),
        pl.BlockSpec((8, _D), lambda i: (0, 0)),
    ],
    out_shape=[
        jax.ShapeDtypeStruct((_NODES, _D), _F32),
        jax.ShapeDtypeStruct((8, _D), _F32),
    ],
)


def _post2b_next_body(p_ref, st_ref, gw_ref, gb_ref, gm_ref, W_ref, dis_ref,
                      out_ref, hd_ref):
    g = _gn_apply(p_ref[...], st_ref, gw_ref, gb_ref, gm_ref)
    out_ref[...] = g
    h = jnp.dot(g, W_ref[...], preferred_element_type=_F32)
    hd_ref[...] = h * dis_ref[...]


_post2b_next = pl.pallas_call(
    _post2b_next_body,
    grid=(_NBLK,),
    in_specs=[
        pl.BlockSpec((_BLK, _D), lambda i: (i, 0)),
        pl.BlockSpec((8, _D), lambda i: (0, 0)),
        pl.BlockSpec((1, _D), lambda i: (0, 0)),
        pl.BlockSpec((1, _D), lambda i: (0, 0)),
        pl.BlockSpec((1, _D), lambda i: (0, 0)),
        pl.BlockSpec((_D, _D), lambda i: (0, 0)),
        pl.BlockSpec((_BLK, 1), lambda i: (i, 0)),
    ],
    out_specs=[
        pl.BlockSpec((_BLK, _D), lambda i: (i, 0)),
        pl.BlockSpec((_BLK, _D), lambda i: (i, 0)),
    ],
    out_shape=[
        jax.ShapeDtypeStruct((_NODES, _D), _F32),
        jax.ShapeDtypeStruct((_NODES, _D), _F32),
    ],
)


def _post2b_body(p_ref, st_ref, gw_ref, gb_ref, gm_ref, out_ref):
    out_ref[...] = _gn_apply(p_ref[...], st_ref, gw_ref, gb_ref, gm_ref)


_post2b = pl.pallas_call(
    _post2b_body,
    grid=(_NBLK,),
    in_specs=[
        pl.BlockSpec((_BLK, _D), lambda i: (i, 0)),
        pl.BlockSpec((8, _D), lambda i: (0, 0)),
        pl.BlockSpec((1, _D), lambda i: (0, 0)),
        pl.BlockSpec((1, _D), lambda i: (0, 0)),
        pl.BlockSpec((1, _D), lambda i: (0, 0)),
    ],
    out_specs=pl.BlockSpec((_BLK, _D), lambda i: (i, 0)),
    out_shape=jax.ShapeDtypeStruct((_NODES, _D), _F32),
)


@jax.jit
def kernel(data, W1, b1, s1, gn1_w, gn1_b, gn1_ms, W2, b2, s2, s3,
           gn3_w, gn3_b, gn3_ms):
    x = data.reshape(_NODES, _D)
    tsc, dis, hd = _edges(data, W1[0])
    ts = tsc.reshape(_NODES // _IC, _IC)

    for l in range(2):
        agg = _scatter(hd, ts)
        f_pre, st = _post1a(x, agg, dis, b1[l].reshape(1, _D),
                            s1[l].reshape(1, 1))
        hd = _post1b(f_pre, st, gn1_w[l].reshape(1, _D),
                     gn1_b[l].reshape(1, _D), gn1_ms[l].reshape(1, _D),
                     W2[l], dis)
        agg2 = _scatter(hd, ts)
        op, st2 = _post2a(x, f_pre, st, gn1_w[l].reshape(1, _D),
                          gn1_b[l].reshape(1, _D), gn1_ms[l].reshape(1, _D),
                          agg2, dis, b2[l].reshape(1, _D),
                          s2[l].reshape(1, 1), s3[l].reshape(1, 1))
        if l == 0:
            x, hd = _post2b_next(op, st2, gn3_w[l].reshape(1, _D),
                                 gn3_b[l].reshape(1, _D),
                                 gn3_ms[l].reshape(1, _D), W1[1], dis)
        else:
            x = _post2b(op, st2, gn3_w[l].reshape(1, _D),
                        gn3_b[l].reshape(1, _D), gn3_ms[l].reshape(1, _D))
    return x


# speedup vs baseline: 31.9341x; 1.0900x over previous
import functools

import jax
import jax.numpy as jnp
from jax import lax
from jax.experimental import pallas as pl
from jax.experimental.pallas import tpu as pltpu
from jax.experimental.pallas import tpu_sc as plsc

_B = 8
_N = 2048
_D = 128
_NODES = _B * _N
_THR = 0.7
_RB = 512
_NBLK = 16
_BLK = _NODES // _NBLK
_NC, _NS = 2, 16
_EPW = _NODES // (_NC * _NS)
_HALF = _NODES // _NC
_TRASH = _HALF
_IC = 128
_F32 = jnp.float32


def _edges_body(d_ref, W_ref, ts_ref, dis_ref, hd_ref):
    b = pl.program_id(0)
    X = d_ref[0]
    nchunks = _N // _RB
    ones_rb = jnp.ones((_RB, 1), _F32)
    ones_n = jnp.ones((_N, 1), _F32)
    iota_n = lax.broadcasted_iota(jnp.int32, (_N, 1), 0).astype(_F32)
    off = (b % (_B // _NC)) * _N

    colp1 = (lax.broadcasted_iota(jnp.int32, (_RB, _N), 1) + 1).astype(_F32)

    def chunk(k, deg_acc):
        Xc = d_ref[0, pl.ds(k * _RB, _RB), :]
        S = lax.dot_general(Xc, X, (((1,), (1,)), ((), ())),
                            preferred_element_type=_F32)
        m = jnp.max(S, axis=1, keepdims=True)
        E = jnp.exp(S - m)
        sm = jnp.sum(E, axis=1, keepdims=True)
        maskf = (E > _THR * sm).astype(_F32)
        amf = jnp.max(maskf * colp1, axis=1)
        ts = jnp.where(amf > 0.5, amf.astype(jnp.int32) - 1 + off, _TRASH)
        ts_ref[0, 0, pl.ds(k * _RB, _RB)] = ts
        return deg_acc + jnp.sum(maskf, axis=0)

    deg = lax.fori_loop(0, nchunks, chunk, jnp.zeros((_N,), _F32))
    dis = lax.rsqrt(1.0 + jnp.transpose(deg.reshape(1, _N)))
    dis_ref[...] = dis
    h = jnp.dot(X, W_ref[...], preferred_element_type=_F32)
    hd_ref[...] = h * dis


_edges = pl.pallas_call(
    _edges_body,
    grid=(_B,),
    in_specs=[
        pl.BlockSpec((1, _N, _D), lambda b: (b, 0, 0)),
        pl.BlockSpec((_D, _D), lambda b: (0, 0)),
    ],
    out_specs=[
        pl.BlockSpec((1, 1, _N), lambda b: (b, 0, 0)),
        pl.BlockSpec((_N, 1), lambda b: (b, 0)),
        pl.BlockSpec((_N, _D), lambda b: (b, 0)),
    ],
    out_shape=[
        jax.ShapeDtypeStruct((_B, 1, _N), jnp.int32),
        jax.ShapeDtypeStruct((_NODES, 1), _F32),
        jax.ShapeDtypeStruct((_NODES, _D), _F32),
    ],
)


def _scatter_body(hd_hbm, ts_hbm, out_hbm, shared, buf0, buf1, idx_v,
                  sem_i, sem0, sem1):
    c = lax.axis_index("c")
    s = lax.axis_index("s")
    base = c * _HALF + s * _EPW
    nch = _EPW // _IC
    bufs = (buf0, buf1)
    sems = (sem0, sem1)
    init = pltpu.async_copy(hd_hbm.at[pl.ds(base, _EPW)],
                            shared.at[pl.ds(s * _EPW, _EPW)], sem_i)
    wid = c * _NS + s
    pltpu.sync_copy(ts_hbm.at[pl.ds(wid * nch, nch)], idx_v)
    gathers = [pltpu.async_copy(hd_hbm.at[pl.ds(base, _IC)], buf0, sem0)]
    init.wait()
    plsc.subcore_barrier()
    for j in range(nch):
        gathers[j].wait()
        if j + 1 < nch:
            gathers.append(
                pltpu.async_copy(hd_hbm.at[pl.ds(base + (j + 1) * _IC, _IC)],
                                 bufs[(j + 1) % 2], sems[(j + 1) % 2]))
        pltpu.sync_copy(bufs[j % 2], shared.at[idx_v.at[j]], add=True)
    plsc.subcore_barrier()
    pltpu.sync_copy(shared.at[pl.ds(s * _EPW, _EPW)],
                    out_hbm.at[pl.ds(base, _EPW)])


@functools.cache
def _get_scatter():
    return pl.kernel(
        _scatter_body,
        out_type=jax.ShapeDtypeStruct((_NODES, _D), _F32),
        mesh=plsc.VectorSubcoreMesh(core_axis_name="c", subcore_axis_name="s",
                                    num_cores=_NC, num_subcores=_NS),
        scratch_types=[
            pltpu.VMEM_SHARED((_HALF + 16, _D), _F32),
            pltpu.VMEM((_IC, _D), _F32),
            pltpu.VMEM((_IC, _D), _F32),
            pltpu.VMEM((_EPW // _IC, _IC), jnp.int32),
            pltpu.SemaphoreType.DMA,
            pltpu.SemaphoreType.DMA,
            pltpu.SemaphoreType.DMA,
        ],
    )


def _scatter(hd, ts):
    return _get_scatter()(hd, ts)


def _msg_norm(h, x_norm, s):
    hn = jnp.sqrt(jnp.sum(h * h, axis=1, keepdims=True))
    return h / jnp.maximum(hn, 1e-12) * x_norm * s


def _accum_stats(i, st_ref, f):
    @pl.when(i == 0)
    def _():
        st_ref[...] = jnp.zeros_like(st_ref)

    st_ref[0:1, :] += jnp.sum(f, axis=0, keepdims=True)
    st_ref[1:2, :] += jnp.sum(f * f, axis=0, keepdims=True)


def _gn_apply(f, st_ref, gw_ref, gb_ref, gm_ref):
    inv_n = 1.0 / _NODES
    mean = st_ref[0:1, :] * inv_n
    mm = mean * gm_ref[...]
    var = st_ref[1:2, :] * inv_n - 2.0 * mm * mean + mm * mm
    o = f - mm
    g = gw_ref[...] * o * lax.rsqrt(var + 1e-5) + gb_ref[...]
    return 0.5 * g * (1.0 + lax.erf(g * 0.7071067811865476))


def _post1a_body(x_ref, agg_ref, dis_ref, b_ref, s_ref, f_ref, st_ref):
    i = pl.program_id(0)
    h = dis_ref[...] * agg_ref[...] + b_ref[...]
    x = x_ref[...]
    xn = jnp.sqrt(jnp.sum(x * x, axis=1, keepdims=True))
    f = _msg_norm(h, xn, s_ref[...]) + x
    f_ref[...] = f
    _accum_stats(i, st_ref, f)


_post1a = pl.pallas_call(
    _post1a_body,
    grid=(_NBLK,),
    in_specs=[
        pl.BlockSpec((_BLK, _D), lambda i: (i, 0)),
        pl.BlockSpec((_BLK, _D), lambda i: (i, 0)),
        pl.BlockSpec((_BLK, 1), lambda i: (i, 0)),
        pl.BlockSpec((1, _D), lambda i: (0, 0)),
        pl.BlockSpec((1, 1), lambda i: (0, 0)),
    ],
    out_specs=[
        pl.BlockSpec((_BLK, _D), lambda i: (i, 0)),
        pl.BlockSpec((8, _D), lambda i: (0, 0)),
    ],
    out_shape=[
        jax.ShapeDtypeStruct((_NODES, _D), _F32),
        jax.ShapeDtypeStruct((8, _D), _F32),
    ],
)


def _post1b_body(f_ref, st_ref, gw_ref, gb_ref, gm_ref, W_ref, dis_ref,
                 hd_ref):
    g = _gn_apply(f_ref[...], st_ref, gw_ref, gb_ref, gm_ref)
    h = jnp.dot(g, W_ref[...], preferred_element_type=_F32)
    hd_ref[...] = h * dis_ref[...]


_post1b = pl.pallas_call(
    _post1b_body,
    grid=(_NBLK,),
    in_specs=[
        pl.BlockSpec((_BLK, _D), lambda i: (i, 0)),
        pl.BlockSpec((8, _D), lambda i: (0, 0)),
        pl.BlockSpec((1, _D), lambda i: (0, 0)),
        pl.BlockSpec((1, _D), lambda i: (0, 0)),
        pl.BlockSpec((1, _D), lambda i: (0, 0)),
        pl.BlockSpec((_D, _D), lambda i: (0, 0)),
        pl.BlockSpec((_BLK, 1), lambda i: (i, 0)),
    ],
    out_specs=pl.BlockSpec((_BLK, _D), lambda i: (i, 0)),
    out_shape=jax.ShapeDtypeStruct((_NODES, _D), _F32),
)


def _post2a_body(x_ref, fp_ref, st1_ref, gw_ref, gb_ref, gm_ref, agg_ref,
                 dis_ref, b_ref, s2_ref, s3_ref, out_ref, st_ref):
    i = pl.program_id(0)
    f = _gn_apply(fp_ref[...], st1_ref, gw_ref, gb_ref, gm_ref)
    h2 = dis_ref[...] * agg_ref[...] + b_ref[...]
    fn = jnp.sqrt(jnp.sum(f * f, axis=1, keepdims=True))
    f2 = _msg_norm(h2, fn, s2_ref[...]) + f
    x = x_ref[...]
    xn = jnp.sqrt(jnp.sum(x * x, axis=1, keepdims=True))
    op = _msg_norm(f2, xn, s3_ref[...]) + x
    out_ref[...] = op
    _accum_stats(i, st_ref, op)


_post2a = pl.pallas_call(
    _post2a_body,
    grid=(_NBLK,),
    in_specs=[
        pl.BlockSpec((_BLK, _D), lambda i: (i, 0)),
        pl.BlockSpec((_BLK, _D), lambda i: (i, 0)),
        pl.BlockSpec((8, _D), lambda i: (0, 0)),
        pl.BlockSpec((1, _D), lambda i: (0, 0)),
        pl.BlockSpec((1, _D), lambda i: (0, 0)),
        pl.BlockSpec((1, _D), lambda i: (0, 0)),
        pl.BlockSpec((_BLK, _D), lambda i: (i, 0)),
        pl.BlockSpec((_BLK, 1), lambda i: (i, 0)),
        pl.BlockSpec((1, _D), lambda i: (0, 0)),
        pl.BlockSpec((1, 1), lambda i: (0, 0)),
        pl.BlockSpec((1, 1), lambda i: (0, 0)),
    ],
    out_specs=[
        pl.BlockSpec((_BLK, _D), lambda i: (i, 0)),
        pl.BlockSpec((8, _D), lambda i: (0, 0)),
    ],
    out_shape=[
        jax.ShapeDtypeStruct((_NODES, _D), _F32),
        jax.ShapeDtypeStruct((8, _D), _F32),
    ],
)


def _post2b_next_body(p_ref, st_ref, gw_ref, gb_ref, gm_ref, W_ref, dis_ref,
                      out_ref, hd_ref):
    g = _gn_apply(p_ref[...], st_ref, gw_ref, gb_ref, gm_ref)
    out_ref[...] = g
    h = jnp.dot(g, W_ref[...], preferred_element_type=_F32)
    hd_ref[...] = h * dis_ref[...]


_post2b_next = pl.pallas_call(
    _post2b_next_body,
    grid=(_NBLK,),
    in_specs=[
        pl.BlockSpec((_BLK, _D), lambda i: (i, 0)),
        pl.BlockSpec((8, _D), lambda i: (0, 0)),
        pl.BlockSpec((1, _D), lambda i: (0, 0)),
        pl.BlockSpec((1, _D), lambda i: (0, 0)),
        pl.BlockSpec((1, _D), lambda i: (0, 0)),
        pl.BlockSpec((_D, _D), lambda i: (0, 0)),
        pl.BlockSpec((_BLK, 1), lambda i: (i, 0)),
    ],
    out_specs=[
        pl.BlockSpec((_BLK, _D), lambda i: (i, 0)),
        pl.BlockSpec((_BLK, _D), lambda i: (i, 0)),
    ],
    out_shape=[
        jax.ShapeDtypeStruct((_NODES, _D), _F32),
        jax.ShapeDtypeStruct((_NODES, _D), _F32),
    ],
)


def _post2b_body(p_ref, st_ref, gw_ref, gb_ref, gm_ref, out_ref):
    out_ref[...] = _gn_apply(p_ref[...], st_ref, gw_ref, gb_ref, gm_ref)


_post2b = pl.pallas_call(
    _post2b_body,
    grid=(_NBLK,),
    in_specs=[
        pl.BlockSpec((_BLK, _D), lambda i: (i, 0)),
        pl.BlockSpec((8, _D), lambda i: (0, 0)),
        pl.BlockSpec((1, _D), lambda i: (0, 0)),
        pl.BlockSpec((1, _D), lambda i: (0, 0)),
        pl.BlockSpec((1, _D), lambda i: (0, 0)),
    ],
    out_specs=pl.BlockSpec((_BLK, _D), lambda i: (i, 0)),
    out_shape=jax.ShapeDtypeStruct((_NODES, _D), _F32),
)


@jax.jit
def kernel(data, W1, b1, s1, gn1_w, gn1_b, gn1_ms, W2, b2, s2, s3,
           gn3_w, gn3_b, gn3_ms):
    x = data.reshape(_NODES, _D)
    tsc, dis, hd = _edges(data, W1[0])
    ts = tsc.reshape(_NODES // _IC, _IC)

    for l in range(2):
        agg = _scatter(hd, ts)
        f_pre, st = _post1a(x, agg, dis, b1[l].reshape(1, _D),
                            s1[l].reshape(1, 1))
        hd = _post1b(f_pre, st, gn1_w[l].reshape(1, _D),
                     gn1_b[l].reshape(1, _D), gn1_ms[l].reshape(1, _D),
                     W2[l], dis)
        agg2 = _scatter(hd, ts)
        op, st2 = _post2a(x, f_pre, st, gn1_w[l].reshape(1, _D),
                          gn1_b[l].reshape(1, _D), gn1_ms[l].reshape(1, _D),
                          agg2, dis, b2[l].reshape(1, _D),
                          s2[l].reshape(1, 1), s3[l].reshape(1, 1))
        if l == 0:
            x, hd = _post2b_next(op, st2, gn3_w[l].reshape(1, _D),
                                 gn3_b[l].reshape(1, _D),
                                 gn3_ms[l].reshape(1, _D), W1[1], dis)
        else:
            x = _post2b(op, st2, gn3_w[l].reshape(1, _D),
                        gn3_b[l].reshape(1, _D), gn3_ms[l].reshape(1, _D))
    return x


# speedup vs baseline: 32.0584x; 1.0039x over previous
import functools

import jax
import jax.numpy as jnp
from jax import lax
from jax.experimental import pallas as pl
from jax.experimental.pallas import tpu as pltpu
from jax.experimental.pallas import tpu_sc as plsc

_B = 8
_N = 2048
_D = 128
_NODES = _B * _N
_THR = 0.7
_RB = 512
_NBLK = 16
_BLK = _NODES // _NBLK
_NC, _NS = 2, 16
_EPW = _NODES // (_NC * _NS)
_HALF = _NODES // _NC
_TRASH = _HALF
_IC = 128
_F32 = jnp.float32


def _edges_body(d_ref, W_ref, ts_ref, dis_ref, hd_ref):
    b = pl.program_id(0)
    X = d_ref[0]
    nchunks = _N // _RB
    ones_rb = jnp.ones((_RB, 1), _F32)
    ones_n = jnp.ones((_N, 1), _F32)
    iota_n = lax.broadcasted_iota(jnp.int32, (_N, 1), 0).astype(_F32)
    off = (b % (_B // _NC)) * _N

    colp1 = (lax.broadcasted_iota(jnp.int32, (_RB, _N), 1) + 1).astype(_F32)

    def chunk(k, deg_acc):
        Xc = d_ref[0, pl.ds(k * _RB, _RB), :]
        S = lax.dot_general(Xc, X, (((1,), (1,)), ((), ())),
                            preferred_element_type=_F32)
        m = jnp.max(S, axis=1, keepdims=True)
        E = jnp.exp(S - m)
        sm = jnp.sum(E, axis=1, keepdims=True)
        maskf = (E > _THR * sm).astype(_F32)
        amf = jnp.max(maskf * colp1, axis=1)
        ts = jnp.where(amf > 0.5, amf.astype(jnp.int32) - 1 + off, _TRASH)
        ts_ref[0, 0, pl.ds(k * _RB, _RB)] = ts
        return deg_acc + jnp.sum(maskf, axis=0)

    deg = lax.fori_loop(0, nchunks, chunk, jnp.zeros((_N,), _F32))
    dis = lax.rsqrt(1.0 + jnp.transpose(deg.reshape(1, _N)))
    dis_ref[...] = dis
    h = jnp.dot(X, W_ref[...], preferred_element_type=_F32)
    hd_ref[...] = h * dis


_edges = pl.pallas_call(
    _edges_body,
    grid=(_B,),
    in_specs=[
        pl.BlockSpec((1, _N, _D), lambda b: (b, 0, 0)),
        pl.BlockSpec((_D, _D), lambda b: (0, 0)),
    ],
    out_specs=[
        pl.BlockSpec((1, 1, _N), lambda b: (b, 0, 0)),
        pl.BlockSpec((_N, 1), lambda b: (b, 0)),
        pl.BlockSpec((_N, _D), lambda b: (b, 0)),
    ],
    out_shape=[
        jax.ShapeDtypeStruct((_B, 1, _N), jnp.int32),
        jax.ShapeDtypeStruct((_NODES, 1), _F32),
        jax.ShapeDtypeStruct((_NODES, _D), _F32),
    ],
)


def _scatter_body(hd_hbm, ts_hbm, out_hbm, shared, buf0, buf1, idx_v,
                  sem_i, sem0, sem1):
    c = lax.axis_index("c")
    s = lax.axis_index("s")
    base = c * _HALF + s * _EPW
    nch = _EPW // _IC
    bufs = (buf0, buf1)
    sems = (sem0, sem1)
    init = pltpu.async_copy(hd_hbm.at[pl.ds(base, _EPW)],
                            shared.at[pl.ds(s * _EPW, _EPW)], sem_i)
    wid = c * _NS + s
    pltpu.sync_copy(ts_hbm.at[pl.ds(wid * nch, nch)], idx_v)
    gathers = [pltpu.async_copy(hd_hbm.at[pl.ds(base, _IC)], buf0, sem0)]
    init.wait()
    plsc.subcore_barrier()
    for j in range(nch):
        gathers[j].wait()
        if j + 1 < nch:
            gathers.append(
                pltpu.async_copy(hd_hbm.at[pl.ds(base + (j + 1) * _IC, _IC)],
                                 bufs[(j + 1) % 2], sems[(j + 1) % 2]))
        pltpu.sync_copy(bufs[j % 2], shared.at[idx_v.at[j]], add=True)
    plsc.subcore_barrier()
    pltpu.sync_copy(shared.at[pl.ds(s * _EPW, _EPW)],
                    out_hbm.at[pl.ds(base, _EPW)])


@functools.cache
def _get_scatter():
    return pl.kernel(
        _scatter_body,
        out_type=jax.ShapeDtypeStruct((_NODES, _D), _F32),
        mesh=plsc.VectorSubcoreMesh(core_axis_name="c", subcore_axis_name="s",
                                    num_cores=_NC, num_subcores=_NS),
        scratch_types=[
            pltpu.VMEM_SHARED((_HALF + 16, _D), _F32),
            pltpu.VMEM((_IC, _D), _F32),
            pltpu.VMEM((_IC, _D), _F32),
            pltpu.VMEM((_EPW // _IC, _IC), jnp.int32),
            pltpu.SemaphoreType.DMA,
            pltpu.SemaphoreType.DMA,
            pltpu.SemaphoreType.DMA,
        ],
    )


def _scatter(hd, ts):
    return _get_scatter()(hd, ts)


def _msg_norm(h, x_norm, s):
    hn = jnp.sqrt(jnp.sum(h * h, axis=1, keepdims=True))
    return h / jnp.maximum(hn, 1e-12) * x_norm * s


def _accum_stats(i, st_ref, f):
    @pl.when(i == 0)
    def _():
        st_ref[...] = jnp.zeros_like(st_ref)

    st_ref[0:1, :] += jnp.sum(f, axis=0, keepdims=True)
    st_ref[1:2, :] += jnp.sum(f * f, axis=0, keepdims=True)


def _gn_apply(f, st_ref, gw_ref, gb_ref, gm_ref):
    inv_n = 1.0 / _NODES
    mean = st_ref[0:1, :] * inv_n
    mm = mean * gm_ref[...]
    var = st_ref[1:2, :] * inv_n - 2.0 * mm * mean + mm * mm
    o = f - mm
    g = gw_ref[...] * o * lax.rsqrt(var + 1e-5) + gb_ref[...]
    return 0.5 * g * (1.0 + lax.erf(g * 0.7071067811865476))


def _post1a_body(x_ref, agg_ref, dis_ref, b_ref, s_ref, f_ref, st_ref):
    i = pl.program_id(0)
    h = dis_ref[...] * agg_ref[...] + b_ref[...]
    x = x_ref[...]
    xn = jnp.sqrt(jnp.sum(x * x, axis=1, keepdims=True))
    f = _msg_norm(h, xn, s_ref[...]) + x
    f_ref[...] = f
    _accum_stats(i, st_ref, f)


_post1a = pl.pallas_call(
    _post1a_body,
    grid=(_NBLK,),
    in_specs=[
        pl.BlockSpec((_BLK, _D), lambda i: (i, 0)),
        pl.BlockSpec((_BLK, _D), lambda i: (i, 0)),
        pl.BlockSpec((_BLK, 1), lambda i: (i, 0)),
        pl.BlockSpec((1, _D), lambda i: (0, 0)),
        pl.BlockSpec((1, 1), lambda i: (0, 0)),
    ],
    out_specs=[
        pl.BlockSpec((_BLK, _D), lambda i: (i, 0)),
        pl.BlockSpec((8, _D), lambda i: (0, 0)),
    ],
    out_shape=[
        jax.ShapeDtypeStruct((_NODES, _D), _F32),
        jax.ShapeDtypeStruct((8, _D), _F32),
    ],
)


def _post1b_body(f_ref, st_ref, gw_ref, gb_ref, gm_ref, W_ref, dis_ref,
                 hd_ref):
    g = _gn_apply(f_ref[...], st_ref, gw_ref, gb_ref, gm_ref)
    h = jnp.dot(g, W_ref[...], preferred_element_type=_F32)
    hd_ref[...] = h * dis_ref[...]


_post1b = pl.pallas_call(
    _post1b_body,
    grid=(_NBLK,),
    in_specs=[
        pl.BlockSpec((_BLK, _D), lambda i: (i, 0)),
        pl.BlockSpec((8, _D), lambda i: (0, 0)),
        pl.BlockSpec((1, _D), lambda i: (0, 0)),
        pl.BlockSpec((1, _D), lambda i: (0, 0)),
        pl.BlockSpec((1, _D), lambda i: (0, 0)),
        pl.BlockSpec((_D, _D), lambda i: (0, 0)),
        pl.BlockSpec((_BLK, 1), lambda i: (i, 0)),
    ],
    out_specs=pl.BlockSpec((_BLK, _D), lambda i: (i, 0)),
    out_shape=jax.ShapeDtypeStruct((_NODES, _D), _F32),
)


def _post2a_body(x_ref, fp_ref, st1_ref, gw_ref, gb_ref, gm_ref, agg_ref,
                 dis_ref, b_ref, s2_ref, s3_ref, out_ref, st_ref):
    i = pl.program_id(0)
    f = _gn_apply(fp_ref[...], st1_ref, gw_ref, gb_ref, gm_ref)
    h2 = dis_ref[...] * agg_ref[...] + b_ref[...]
    fn = jnp.sqrt(jnp.sum(f * f, axis=1, keepdims=True))
    f2 = _msg_norm(h2, fn, s2_ref[...]) + f
    x = x_ref[...]
    xn = jnp.sqrt(jnp.sum(x * x, axis=1, keepdims=True))
    op = _msg_norm(f2, xn, s3_ref[...]) + x
    out_ref[...] = op
    _accum_stats(i, st_ref, op)


_post2a = pl.pallas_call(
    _post2a_body,
    grid=(_NBLK,),
    in_specs=[
        pl.BlockSpec((_BLK, _D), lambda i: (i, 0)),
        pl.BlockSpec((_BLK, _D), lambda i: (i, 0)),
        pl.BlockSpec((8, _D), lambda i: (0, 0)),
        pl.BlockSpec((1, _D), lambda i: (0, 0)),
        pl.BlockSpec((1, _D), lambda i: (0, 0)),
        pl.BlockSpec((1, _D), lambda i: (0, 0)),
        pl.BlockSpec((_BLK, _D), lambda i: (i, 0)),
        pl.BlockSpec((_BLK, 1), lambda i: (i, 0)),
        pl.BlockSpec((1, _D), lambda i: (0, 0)),
        pl.BlockSpec((1, 1), lambda i: (0, 0)),
        pl.BlockSpec((1, 1), lambda i: (0, 0)),
    ],
    out_specs=[
        pl.BlockSpec((_BLK, _D), lambda i: (i, 0)),
        pl.BlockSpec((8, _D), lambda i: (0, 0)),
    ],
    out_shape=[
        jax.ShapeDtypeStruct((_NODES, _D), _F32),
        jax.ShapeDtypeStruct((8, _D), _F32),
    ],
)


def _post2b_next_body(p_ref, st_ref, gw_ref, gb_ref, gm_ref, W_ref, dis_ref,
                      hd_ref):
    g = _gn_apply(p_ref[...], st_ref, gw_ref, gb_ref, gm_ref)
    h = jnp.dot(g, W_ref[...], preferred_element_type=_F32)
    hd_ref[...] = h * dis_ref[...]


_post2b_next = pl.pallas_call(
    _post2b_next_body,
    grid=(_NBLK,),
    in_specs=[
        pl.BlockSpec((_BLK, _D), lambda i: (i, 0)),
        pl.BlockSpec((8, _D), lambda i: (0, 0)),
        pl.BlockSpec((1, _D), lambda i: (0, 0)),
        pl.BlockSpec((1, _D), lambda i: (0, 0)),
        pl.BlockSpec((1, _D), lambda i: (0, 0)),
        pl.BlockSpec((_D, _D), lambda i: (0, 0)),
        pl.BlockSpec((_BLK, 1), lambda i: (i, 0)),
    ],
    out_specs=pl.BlockSpec((_BLK, _D), lambda i: (i, 0)),
    out_shape=jax.ShapeDtypeStruct((_NODES, _D), _F32),
)


def _post1a_r_body(p_ref, stp_ref, pw_ref, pb_ref, pm_ref, agg_ref, dis_ref,
                   b_ref, s_ref, f_ref, st_ref):
    i = pl.program_id(0)
    x = _gn_apply(p_ref[...], stp_ref, pw_ref, pb_ref, pm_ref)
    h = dis_ref[...] * agg_ref[...] + b_ref[...]
    xn = jnp.sqrt(jnp.sum(x * x, axis=1, keepdims=True))
    f = _msg_norm(h, xn, s_ref[...]) + x
    f_ref[...] = f
    _accum_stats(i, st_ref, f)


_post1a_r = pl.pallas_call(
    _post1a_r_body,
    grid=(_NBLK,),
    in_specs=[
        pl.BlockSpec((_BLK, _D), lambda i: (i, 0)),
        pl.BlockSpec((8, _D), lambda i: (0, 0)),
        pl.BlockSpec((1, _D), lambda i: (0, 0)),
        pl.BlockSpec((1, _D), lambda i: (0, 0)),
        pl.BlockSpec((1, _D), lambda i: (0, 0)),
        pl.BlockSpec((_BLK, _D), lambda i: (i, 0)),
        pl.BlockSpec((_BLK, 1), lambda i: (i, 0)),
        pl.BlockSpec((1, _D), lambda i: (0, 0)),
        pl.BlockSpec((1, 1), lambda i: (0, 0)),
    ],
    out_specs=[
        pl.BlockSpec((_BLK, _D), lambda i: (i, 0)),
        pl.BlockSpec((8, _D), lambda i: (0, 0)),
    ],
    out_shape=[
        jax.ShapeDtypeStruct((_NODES, _D), _F32),
        jax.ShapeDtypeStruct((8, _D), _F32),
    ],
)


def _post2a_r_body(p_ref, stp_ref, pw_ref, pb_ref, pm_ref, fp_ref, st1_ref,
                   gw_ref, gb_ref, gm_ref, agg_ref, dis_ref, b_ref, s2_ref,
                   s3_ref, out_ref, st_ref):
    i = pl.program_id(0)
    x = _gn_apply(p_ref[...], stp_ref, pw_ref, pb_ref, pm_ref)
    f = _gn_apply(fp_ref[...], st1_ref, gw_ref, gb_ref, gm_ref)
    h2 = dis_ref[...] * agg_ref[...] + b_ref[...]
    fn = jnp.sqrt(jnp.sum(f * f, axis=1, keepdims=True))
    f2 = _msg_norm(h2, fn, s2_ref[...]) + f
    xn = jnp.sqrt(jnp.sum(x * x, axis=1, keepdims=True))
    op = _msg_norm(f2, xn, s3_ref[...]) + x
    out_ref[...] = op
    _accum_stats(i, st_ref, op)


_post2a_r = pl.pallas_call(
    _post2a_r_body,
    grid=(_NBLK,),
    in_specs=[
        pl.BlockSpec((_BLK, _D), lambda i: (i, 0)),
        pl.BlockSpec((8, _D), lambda i: (0, 0)),
        pl.BlockSpec((1, _D), lambda i: (0, 0)),
        pl.BlockSpec((1, _D), lambda i: (0, 0)),
        pl.BlockSpec((1, _D), lambda i: (0, 0)),
        pl.BlockSpec((_BLK, _D), lambda i: (i, 0)),
        pl.BlockSpec((8, _D), lambda i: (0, 0)),
        pl.BlockSpec((1, _D), lambda i: (0, 0)),
        pl.BlockSpec((1, _D), lambda i: (0, 0)),
        pl.BlockSpec((1, _D), lambda i: (0, 0)),
        pl.BlockSpec((_BLK, _D), lambda i: (i, 0)),
        pl.BlockSpec((_BLK, 1), lambda i: (i, 0)),
        pl.BlockSpec((1, _D), lambda i: (0, 0)),
        pl.BlockSpec((1, 1), lambda i: (0, 0)),
        pl.BlockSpec((1, 1), lambda i: (0, 0)),
    ],
    out_specs=[
        pl.BlockSpec((_BLK, _D), lambda i: (i, 0)),
        pl.BlockSpec((8, _D), lambda i: (0, 0)),
    ],
    out_shape=[
        jax.ShapeDtypeStruct((_NODES, _D), _F32),
        jax.ShapeDtypeStruct((8, _D), _F32),
    ],
)


def _post2b_body(p_ref, st_ref, gw_ref, gb_ref, gm_ref, out_ref):
    out_ref[...] = _gn_apply(p_ref[...], st_ref, gw_ref, gb_ref, gm_ref)


_post2b = pl.pallas_call(
    _post2b_body,
    grid=(_NBLK,),
    in_specs=[
        pl.BlockSpec((_BLK, _D), lambda i: (i, 0)),
        pl.BlockSpec((8, _D), lambda i: (0, 0)),
        pl.BlockSpec((1, _D), lambda i: (0, 0)),
        pl.BlockSpec((1, _D), lambda i: (0, 0)),
        pl.BlockSpec((1, _D), lambda i: (0, 0)),
    ],
    out_specs=pl.BlockSpec((_BLK, _D), lambda i: (i, 0)),
    out_shape=jax.ShapeDtypeStruct((_NODES, _D), _F32),
)


@jax.jit
def kernel(data, W1, b1, s1, gn1_w, gn1_b, gn1_ms, W2, b2, s2, s3,
           gn3_w, gn3_b, gn3_ms):
    x = data.reshape(_NODES, _D)
    tsc, dis, hd = _edges(data, W1[0])
    ts = tsc.reshape(_NODES // _IC, _IC)

    r1 = lambda a: a.reshape(1, _D)
    rs = lambda a: a.reshape(1, 1)

    agg = _scatter(hd, ts)
    f_pre, st = _post1a(x, agg, dis, r1(b1[0]), rs(s1[0]))
    hd = _post1b(f_pre, st, r1(gn1_w[0]), r1(gn1_b[0]), r1(gn1_ms[0]),
                 W2[0], dis)
    agg2 = _scatter(hd, ts)
    op, st2 = _post2a(x, f_pre, st, r1(gn1_w[0]), r1(gn1_b[0]),
                      r1(gn1_ms[0]), agg2, dis, r1(b2[0]), rs(s2[0]),
                      rs(s3[0]))
    hd = _post2b_next(op, st2, r1(gn3_w[0]), r1(gn3_b[0]), r1(gn3_ms[0]),
                      W1[1], dis)

    agg = _scatter(hd, ts)
    f_pre2, stb = _post1a_r(op, st2, r1(gn3_w[0]), r1(gn3_b[0]),
                            r1(gn3_ms[0]), agg, dis, r1(b1[1]), rs(s1[1]))
    hd = _post1b(f_pre2, stb, r1(gn1_w[1]), r1(gn1_b[1]), r1(gn1_ms[1]),
                 W2[1], dis)
    agg2 = _scatter(hd, ts)
    op2, st3 = _post2a_r(op, st2, r1(gn3_w[0]), r1(gn3_b[0]), r1(gn3_ms[0]),
                         f_pre2, stb, r1(gn1_w[1]), r1(gn1_b[1]),
                         r1(gn1_ms[1]), agg2, dis, r1(b2[1]), rs(s2[1]),
                         rs(s3[1]))
    return _post2b(op2, st3, r1(gn3_w[1]), r1(gn3_b[1]), r1(gn3_ms[1]))
